# Initial kernel scaffold; baseline (speedup 1.0000x reference)
import jax
import jax.numpy as jnp
from jax.experimental import pallas as pl


def kernel(atomic_numbers, edge_index, edge_vec, batch_ids, atom_table, W_rbf, Wg, Wv, We, gamma, W1, W2, F1, F2):
    raise NotImplementedError("write your pallas kernel here")


# speedup vs baseline: 1.8420x; 1.8420x over previous
import functools

import jax
import jax.numpy as jnp
from jax import lax
from jax.experimental import pallas as pl
from jax.experimental.pallas import tpu as pltpu
from jax.experimental.pallas import tpu_sc as plsc

N = 10000
E = 320000
G = 16
D = 128
R = 7
L = 6
CUT = 6.0

NC = 2
NS = 16
NW = NC * NS
CH = 80
FPAD = 128


def _gather_body(table, src, dst, hs_out, hd_out, idx_v, rows_v, sem):
    wid = lax.axis_index("c") * NS + lax.axis_index("s")
    per_w = E // NW
    base = wid * per_w

    def one(idx_hbm, out_hbm):
        def step(i, _):
            off = base + i * CH
            pltpu.sync_copy(idx_hbm.at[pl.ds(off, CH)], idx_v)
            pltpu.async_copy(table.at[idx_v], rows_v, sem).wait()
            pltpu.sync_copy(rows_v, out_hbm.at[pl.ds(off, CH), :])
            return 0
        lax.fori_loop(0, per_w // CH, step, 0)

    one(src, hs_out)
    one(dst, hd_out)


@functools.lru_cache(maxsize=None)
def _sc_kernels():
    mesh = plsc.VectorSubcoreMesh(core_axis_name="c", subcore_axis_name="s",
                                  num_cores=NC, num_subcores=NS)
    gather = pl.kernel(
        _gather_body,
        out_type=(
            jax.ShapeDtypeStruct((E, D), jnp.float32),
            jax.ShapeDtypeStruct((E, D), jnp.float32),
        ),
        mesh=mesh,
        scratch_types=[
            pltpu.VMEM((CH,), jnp.int32),
            pltpu.VMEM((CH, D), jnp.float32),
            pltpu.SemaphoreType.DMA,
        ],
    )
    scatter = pl.kernel(
        _scatter_body,
        out_type=jax.ShapeDtypeStruct((NC, N, D), jnp.float32),
        mesh=mesh,
        scratch_types=[
            pltpu.VMEM_SHARED((N, D), jnp.float32),
            pltpu.VMEM((CH,), jnp.int32),
            pltpu.VMEM((CH, D), jnp.float32),
        ],
    )
    fscatter = pl.kernel(
        _fscatter_body,
        out_type=jax.ShapeDtypeStruct((NC, N, FPAD), jnp.float32),
        mesh=mesh,
        scratch_types=[
            pltpu.VMEM_SHARED((N, FPAD), jnp.float32),
            pltpu.VMEM((CH,), jnp.int32),
            pltpu.VMEM((CH, FPAD), jnp.float32),
        ],
    )
    return gather, scatter, fscatter


def _sc_gather(table, src, dst):
    return _sc_kernels()[0](table, src, dst)


def _scatter_body(msg, dst, zeros, out, acc, idx_v, rows_v):
    cid = lax.axis_index("c")
    sid = lax.axis_index("s")

    @pl.when(sid == 0)
    def _():
        pltpu.sync_copy(zeros, acc)

    plsc.subcore_barrier()

    per_w = (E // NC) // NS
    base = cid * (E // NC) + sid * per_w

    def step(i, _):
        off = base + i * CH
        pltpu.sync_copy(dst.at[pl.ds(off, CH)], idx_v)
        pltpu.sync_copy(msg.at[pl.ds(off, CH), :], rows_v)
        pltpu.sync_copy(rows_v, acc.at[idx_v], add=True)
        return 0
    lax.fori_loop(0, per_w // CH, step, 0)

    plsc.subcore_barrier()

    @pl.when(sid == 0)
    def _():
        pltpu.sync_copy(acc, out.at[cid])


def _sc_scatter(msg, dst, zeros):
    return _sc_kernels()[1](msg, dst, zeros)


def _fscatter_body(fv, fvn, src, dst, zeros, out, acc, idx_v, rows_v):
    cid = lax.axis_index("c")
    sid = lax.axis_index("s")

    @pl.when(sid == 0)
    def _():
        pltpu.sync_copy(zeros, acc)

    plsc.subcore_barrier()

    per_w = (E // NC) // NS
    base = cid * (E // NC) + sid * per_w

    def step(i, _):
        off = base + i * CH
        pltpu.sync_copy(dst.at[pl.ds(off, CH)], idx_v)
        pltpu.sync_copy(fv.at[pl.ds(off, CH), :], rows_v)
        pltpu.sync_copy(rows_v, acc.at[idx_v], add=True)
        pltpu.sync_copy(src.at[pl.ds(off, CH)], idx_v)
        pltpu.sync_copy(fvn.at[pl.ds(off, CH), :], rows_v)
        pltpu.sync_copy(rows_v, acc.at[idx_v], add=True)
        return 0
    lax.fori_loop(0, per_w // CH, step, 0)

    plsc.subcore_barrier()

    @pl.when(sid == 0)
    def _():
        pltpu.sync_copy(acc, out.at[cid])


def _sc_fscatter(fv, fvn, src, dst, zeros):
    return _sc_kernels()[2](fv, fvn, src, dst, zeros)


B_EDGE = 2000
B_NODE = 2000


def _edge_init_body(ev_ref, wr_ref, e0_ref, sm_ref, w_ref):
    ev = ev_ref[...]
    d2 = jnp.sum(ev * ev, axis=1, keepdims=True)
    d = jnp.sqrt(d2 + 1e-12)
    u = d / CUT
    u2 = u * u
    u4 = u2 * u2
    u8 = u4 * u4
    env = 1.0 + (-45.0) * u8 + 80.0 * u8 * u + (-36.0) * u8 * u2
    smooth = jnp.where(u < 1.0, env, 0.0)
    dinv = 1.0 / (d + 1e-8)
    k = lax.broadcasted_iota(jnp.int32, (ev.shape[0], R), 1).astype(jnp.float32) + 1.0
    s = jnp.sin(k * (jnp.pi * u))
    rbf = (jnp.sqrt(2.0 / CUT) * dinv * smooth) * s
    acc = jnp.zeros((ev.shape[0], D), jnp.float32)
    for r in range(R):
        acc = acc + rbf[:, r:r + 1] * wr_ref[r:r + 1, :]
    e0_ref[...] = acc
    sm_ref[...] = smooth
    w_ref[...] = ev * dinv * smooth


def _edge_init(edge_vec, W_rbf):
    return pl.pallas_call(
        _edge_init_body,
        grid=(E // B_EDGE,),
        in_specs=[
            pl.BlockSpec((B_EDGE, 3), lambda i: (i, 0)),
            pl.BlockSpec((R, D), lambda i: (0, 0)),
        ],
        out_specs=[
            pl.BlockSpec((B_EDGE, D), lambda i: (i, 0)),
            pl.BlockSpec((B_EDGE, 1), lambda i: (i, 0)),
            pl.BlockSpec((B_EDGE, 3), lambda i: (i, 0)),
        ],
        out_shape=[
            jax.ShapeDtypeStruct((E, D), jnp.float32),
            jax.ShapeDtypeStruct((E, 1), jnp.float32),
            jax.ShapeDtypeStruct((E, 3), jnp.float32),
        ],
    )(edge_vec, W_rbf)


def _embed_body(an_ref, tab_ref, h_ref):
    an = an_ref[...]
    oh = (an == lax.broadcasted_iota(jnp.int32, (an.shape[0], 95), 1))
    h_ref[...] = jnp.dot(oh.astype(jnp.float32), tab_ref[...],
                         preferred_element_type=jnp.float32)


def _embed(atomic_numbers2d, atom_table):
    return pl.pallas_call(
        _embed_body,
        grid=(N // B_NODE,),
        in_specs=[
            pl.BlockSpec((B_NODE, 1), lambda i: (i, 0)),
            pl.BlockSpec((95, D), lambda i: (0, 0)),
        ],
        out_specs=pl.BlockSpec((B_NODE, D), lambda i: (i, 0)),
        out_shape=jax.ShapeDtypeStruct((N, D), jnp.float32),
    )(atomic_numbers2d, atom_table)


def _combine_body(hs_ref, hd_ref, e_ref, sm_ref, wh_ref, wl_ref, msg_ref, enew_ref):
    e = e_ref[...]
    x = jnp.concatenate([hs_ref[...], hd_ref[...], e], axis=1)
    xh = x.astype(jnp.bfloat16)
    xl = (x - xh.astype(jnp.float32)).astype(jnp.bfloat16)
    wh = wh_ref[...]
    y = (jnp.dot(xh, wh, preferred_element_type=jnp.float32)
         + jnp.dot(xl, wh, preferred_element_type=jnp.float32)
         + jnp.dot(xh, wl_ref[...], preferred_element_type=jnp.float32))
    yg = y[:, :D]
    yv = y[:, D:2 * D]
    ye = y[:, 2 * D:]
    sm = sm_ref[...]
    msg_ref[...] = jax.nn.silu(yg) * yv * sm
    enew_ref[...] = e + jax.nn.silu(ye)


def _combine(hs, hd, e, smooth, Wh, Wlo):
    return pl.pallas_call(
        _combine_body,
        grid=(E // B_EDGE,),
        in_specs=[
            pl.BlockSpec((B_EDGE, D), lambda i: (i, 0)),
            pl.BlockSpec((B_EDGE, D), lambda i: (i, 0)),
            pl.BlockSpec((B_EDGE, D), lambda i: (i, 0)),
            pl.BlockSpec((B_EDGE, 1), lambda i: (i, 0)),
            pl.BlockSpec((3 * D, 3 * D), lambda i: (0, 0)),
            pl.BlockSpec((3 * D, 3 * D), lambda i: (0, 0)),
        ],
        out_specs=[
            pl.BlockSpec((B_EDGE, D), lambda i: (i, 0)),
            pl.BlockSpec((B_EDGE, D), lambda i: (i, 0)),
        ],
        out_shape=[
            jax.ShapeDtypeStruct((E, D), jnp.float32),
            jax.ShapeDtypeStruct((E, D), jnp.float32),
        ],
    )(hs, hd, e, smooth, Wh, Wlo)


def _update_body(h_ref, p_ref, hn_ref):
    hn_ref[...] = h_ref[...] + p_ref[0] + p_ref[1]


def _update(h, parts):
    return pl.pallas_call(
        _update_body,
        grid=(N // B_NODE,),
        in_specs=[
            pl.BlockSpec((B_NODE, D), lambda i: (i, 0)),
            pl.BlockSpec((NC, B_NODE, D), lambda i: (0, i, 0)),
        ],
        out_specs=pl.BlockSpec((B_NODE, D), lambda i: (i, 0)),
        out_shape=jax.ShapeDtypeStruct((N, D), jnp.float32),
    )(h, parts)


def _head_body(h_ref, bid_ref, gamma_ref, w1_ref, w2_ref, out_ref, esum, csum):
    i = pl.program_id(0)

    @pl.when(i == 0)
    def _():
        esum[...] = jnp.zeros_like(esum)
        csum[...] = jnp.zeros_like(csum)

    h = h_ref[...]
    ms = jnp.mean(h * h, axis=1, keepdims=True)
    hn = h * lax.rsqrt(ms + 1e-6) * gamma_ref[0:1, :]
    t = jax.nn.silu(jnp.dot(hn, w1_ref[...], preferred_element_type=jnp.float32))
    ea = jnp.sum(t * w2_ref[0:1, :], axis=1, keepdims=True)
    mask = (bid_ref[...] == lax.broadcasted_iota(jnp.int32, (h.shape[0], G), 1))
    esum[...] += jnp.sum(jnp.where(mask, ea, 0.0), axis=0, keepdims=True)
    csum[...] += jnp.sum(mask.astype(jnp.float32), axis=0, keepdims=True)

    @pl.when(i == pl.num_programs(0) - 1)
    def _():
        out_ref[...] = esum[...] / jnp.maximum(csum[...], 1.0)


def _head(h, batch_ids2d, gamma2d, W1, W2row):
    return pl.pallas_call(
        _head_body,
        grid=(N // B_NODE,),
        in_specs=[
            pl.BlockSpec((B_NODE, D), lambda i: (i, 0)),
            pl.BlockSpec((B_NODE, 1), lambda i: (i, 0)),
            pl.BlockSpec((1, D), lambda i: (0, 0)),
            pl.BlockSpec((D, D), lambda i: (0, 0)),
            pl.BlockSpec((1, D), lambda i: (0, 0)),
        ],
        out_specs=pl.BlockSpec((1, G), lambda i: (0, 0)),
        out_shape=jax.ShapeDtypeStruct((1, G), jnp.float32),
        scratch_shapes=[
            pltpu.VMEM((1, G), jnp.float32),
            pltpu.VMEM((1, G), jnp.float32),
        ],
    )(h, batch_ids2d, gamma2d, W1, W2row)


def _force_body(e_ref, w_ref, f1_ref, f2_ref, fv_ref, fvn_ref):
    t = jax.nn.silu(jnp.dot(e_ref[...], f1_ref[...],
                            preferred_element_type=jnp.float32))
    fs = jnp.sum(t * f2_ref[0:1, :], axis=1, keepdims=True)
    fvec = fs * w_ref[...]
    z = jnp.zeros((fvec.shape[0], FPAD - 3), jnp.float32)
    fv = jnp.concatenate([fvec, z], axis=1)
    fv_ref[...] = fv
    fvn_ref[...] = -fv


def _force_head(e, w, F1, F2row):
    return pl.pallas_call(
        _force_body,
        grid=(E // B_EDGE,),
        in_specs=[
            pl.BlockSpec((B_EDGE, D), lambda i: (i, 0)),
            pl.BlockSpec((B_EDGE, 3), lambda i: (i, 0)),
            pl.BlockSpec((D, D), lambda i: (0, 0)),
            pl.BlockSpec((1, D), lambda i: (0, 0)),
        ],
        out_specs=[
            pl.BlockSpec((B_EDGE, FPAD), lambda i: (i, 0)),
            pl.BlockSpec((B_EDGE, FPAD), lambda i: (i, 0)),
        ],
        out_shape=[
            jax.ShapeDtypeStruct((E, FPAD), jnp.float32),
            jax.ShapeDtypeStruct((E, FPAD), jnp.float32),
        ],
    )(e, w, F1, F2row)


def _fassemble_body(p_ref, f_ref):
    f_ref[...] = (p_ref[0] + p_ref[1])[:, :3]


def _fassemble(parts):
    return pl.pallas_call(
        _fassemble_body,
        grid=(N // B_NODE,),
        in_specs=[pl.BlockSpec((NC, B_NODE, FPAD), lambda i: (0, i, 0))],
        out_specs=pl.BlockSpec((B_NODE, 3), lambda i: (i, 0)),
        out_shape=jax.ShapeDtypeStruct((N, 3), jnp.float32),
    )(parts)


def kernel(atomic_numbers, edge_index, edge_vec, batch_ids, atom_table,
           W_rbf, Wg, Wv, We, gamma, W1, W2, F1, F2):
    src = edge_index[0]
    dst = edge_index[1]

    e, smooth, w = _edge_init(edge_vec, W_rbf)
    h = _embed(atomic_numbers.astype(jnp.int32).reshape(N, 1), atom_table)

    Wcat = jnp.concatenate([Wg, Wv, We], axis=2)
    Wcat_h = Wcat.astype(jnp.bfloat16)
    Wcat_l = (Wcat - Wcat_h.astype(jnp.float32)).astype(jnp.bfloat16)

    zeros_nd = jnp.zeros((N, D), jnp.float32)
    for l in range(L):
        hs, hd = _sc_gather(h, src, dst)
        msg, e = _combine(hs, hd, e, smooth, Wcat_h[l], Wcat_l[l])
        parts = _sc_scatter(msg, dst, zeros_nd)
        h = _update(h, parts)

    energy = _head(h, batch_ids.astype(jnp.int32).reshape(N, 1),
                   gamma.reshape(1, D), W1, W2.reshape(1, D))[0]

    fv, fvn = _force_head(e, w, F1, F2.reshape(1, D))
    zeros_n8 = jnp.zeros((N, FPAD), jnp.float32)
    fparts = _sc_fscatter(fv, fvn, src, dst, zeros_n8)
    forces = _fassemble(fparts)

    return (energy, forces)


# speedup vs baseline: 2.2781x; 1.2368x over previous
import functools

import jax
import jax.numpy as jnp
from jax import lax
from jax.experimental import pallas as pl
from jax.experimental.pallas import tpu as pltpu
from jax.experimental.pallas import tpu_sc as plsc

N = 10000
E = 320000
G = 16
D = 128
R = 7
L = 6
CUT = 6.0

NC = 2
NS = 16
NW = NC * NS
CHG = 400
CH = 200
FPAD = 128


def _gather_body(table, src, dst, hs_out, hd_out, idx_v, rows_v, sem):
    wid = lax.axis_index("c") * NS + lax.axis_index("s")
    per_w = E // NW
    base = wid * per_w

    def one(idx_hbm, out_hbm):
        def step(i, _):
            off = base + i * CHG
            pltpu.sync_copy(idx_hbm.at[pl.ds(off, CHG)], idx_v)
            pltpu.async_copy(table.at[idx_v], rows_v, sem).wait()
            pltpu.sync_copy(rows_v, out_hbm.at[pl.ds(off, CHG), :])
            return 0
        lax.fori_loop(0, per_w // CHG, step, 0)

    one(src, hs_out)
    one(dst, hd_out)


@functools.lru_cache(maxsize=None)
def _sc_kernels():
    mesh = plsc.VectorSubcoreMesh(core_axis_name="c", subcore_axis_name="s",
                                  num_cores=NC, num_subcores=NS)
    gather = pl.kernel(
        _gather_body,
        out_type=(
            jax.ShapeDtypeStruct((E, D), jnp.float32),
            jax.ShapeDtypeStruct((E, D), jnp.float32),
        ),
        mesh=mesh,
        scratch_types=[
            pltpu.VMEM((CHG,), jnp.int32),
            pltpu.VMEM((CHG, D), jnp.float32),
            pltpu.SemaphoreType.DMA,
        ],
    )
    scatter = pl.kernel(
        _scatter_body,
        out_type=jax.ShapeDtypeStruct((NC, N, D), jnp.float32),
        mesh=mesh,
        scratch_types=[
            pltpu.VMEM_SHARED((N, D), jnp.float32),
            pltpu.VMEM((CH,), jnp.int32),
            pltpu.VMEM((CH, D), jnp.float32),
        ],
    )
    fscatter = pl.kernel(
        _fscatter_body,
        out_type=jax.ShapeDtypeStruct((NC, N, FPAD), jnp.float32),
        mesh=mesh,
        scratch_types=[
            pltpu.VMEM_SHARED((N, FPAD), jnp.float32),
            pltpu.VMEM((CH,), jnp.int32),
            pltpu.VMEM((CH, FPAD), jnp.float32),
        ],
    )
    return gather, scatter, fscatter


def _sc_gather(table, src, dst):
    return _sc_kernels()[0](table, src, dst)


def _scatter_body(msg, dst, zeros, out, acc, idx_v, rows_v):
    cid = lax.axis_index("c")
    sid = lax.axis_index("s")

    @pl.when(sid == 0)
    def _():
        pltpu.sync_copy(zeros, acc)

    plsc.subcore_barrier()

    per_w = (E // NC) // NS
    base = cid * (E // NC) + sid * per_w

    def step(i, _):
        off = base + i * CH
        pltpu.sync_copy(dst.at[pl.ds(off, CH)], idx_v)
        pltpu.sync_copy(msg.at[pl.ds(off, CH), :], rows_v)
        pltpu.sync_copy(rows_v, acc.at[idx_v], add=True)
        return 0
    lax.fori_loop(0, per_w // CH, step, 0)

    plsc.subcore_barrier()

    @pl.when(sid == 0)
    def _():
        pltpu.sync_copy(acc, out.at[cid])


def _sc_scatter(msg, dst, zeros):
    return _sc_kernels()[1](msg, dst, zeros)


def _fscatter_body(fv, fvn, src, dst, zeros, out, acc, idx_v, rows_v):
    cid = lax.axis_index("c")
    sid = lax.axis_index("s")

    @pl.when(sid == 0)
    def _():
        pltpu.sync_copy(zeros, acc)

    plsc.subcore_barrier()

    per_w = (E // NC) // NS
    base = cid * (E // NC) + sid * per_w

    def step(i, _):
        off = base + i * CH
        pltpu.sync_copy(dst.at[pl.ds(off, CH)], idx_v)
        pltpu.sync_copy(fv.at[pl.ds(off, CH), :], rows_v)
        pltpu.sync_copy(rows_v, acc.at[idx_v], add=True)
        pltpu.sync_copy(src.at[pl.ds(off, CH)], idx_v)
        pltpu.sync_copy(fvn.at[pl.ds(off, CH), :], rows_v)
        pltpu.sync_copy(rows_v, acc.at[idx_v], add=True)
        return 0
    lax.fori_loop(0, per_w // CH, step, 0)

    plsc.subcore_barrier()

    @pl.when(sid == 0)
    def _():
        pltpu.sync_copy(acc, out.at[cid])


def _sc_fscatter(fv, fvn, src, dst, zeros):
    return _sc_kernels()[2](fv, fvn, src, dst, zeros)


B_EDGE = 2000
B_NODE = 2000


def _edge_init_body(ev_ref, wr_ref, e0_ref, sm_ref, w_ref):
    ev = ev_ref[...]
    d2 = jnp.sum(ev * ev, axis=1, keepdims=True)
    d = jnp.sqrt(d2 + 1e-12)
    u = d / CUT
    u2 = u * u
    u4 = u2 * u2
    u8 = u4 * u4
    env = 1.0 + (-45.0) * u8 + 80.0 * u8 * u + (-36.0) * u8 * u2
    smooth = jnp.where(u < 1.0, env, 0.0)
    dinv = 1.0 / (d + 1e-8)
    k = lax.broadcasted_iota(jnp.int32, (ev.shape[0], R), 1).astype(jnp.float32) + 1.0
    s = jnp.sin(k * (jnp.pi * u))
    rbf = (jnp.sqrt(2.0 / CUT) * dinv * smooth) * s
    acc = jnp.zeros((ev.shape[0], D), jnp.float32)
    for r in range(R):
        acc = acc + rbf[:, r:r + 1] * wr_ref[r:r + 1, :]
    e0_ref[...] = acc
    sm_ref[...] = smooth
    w_ref[...] = ev * dinv * smooth


def _edge_init(edge_vec, W_rbf):
    return pl.pallas_call(
        _edge_init_body,
        grid=(E // B_EDGE,),
        in_specs=[
            pl.BlockSpec((B_EDGE, 3), lambda i: (i, 0)),
            pl.BlockSpec((R, D), lambda i: (0, 0)),
        ],
        out_specs=[
            pl.BlockSpec((B_EDGE, D), lambda i: (i, 0)),
            pl.BlockSpec((B_EDGE, 1), lambda i: (i, 0)),
            pl.BlockSpec((B_EDGE, 3), lambda i: (i, 0)),
        ],
        out_shape=[
            jax.ShapeDtypeStruct((E, D), jnp.float32),
            jax.ShapeDtypeStruct((E, 1), jnp.float32),
            jax.ShapeDtypeStruct((E, 3), jnp.float32),
        ],
    )(edge_vec, W_rbf)


def _embed_body(an_ref, tab_ref, h_ref):
    an = an_ref[...]
    oh = (an == lax.broadcasted_iota(jnp.int32, (an.shape[0], 95), 1))
    h_ref[...] = jnp.dot(oh.astype(jnp.float32), tab_ref[...],
                         preferred_element_type=jnp.float32)


def _embed(atomic_numbers2d, atom_table):
    return pl.pallas_call(
        _embed_body,
        grid=(N // B_NODE,),
        in_specs=[
            pl.BlockSpec((B_NODE, 1), lambda i: (i, 0)),
            pl.BlockSpec((95, D), lambda i: (0, 0)),
        ],
        out_specs=pl.BlockSpec((B_NODE, D), lambda i: (i, 0)),
        out_shape=jax.ShapeDtypeStruct((N, D), jnp.float32),
    )(atomic_numbers2d, atom_table)


def _combine_body(hs_ref, hd_ref, e_ref, sm_ref, wh_ref, wl_ref, msg_ref, enew_ref):
    e = e_ref[...]
    x = jnp.concatenate([hs_ref[...], hd_ref[...], e], axis=1)
    xh = x.astype(jnp.bfloat16)
    xl = (x - xh.astype(jnp.float32)).astype(jnp.bfloat16)
    wh = wh_ref[...]
    y = (jnp.dot(xh, wh, preferred_element_type=jnp.float32)
         + jnp.dot(xl, wh, preferred_element_type=jnp.float32)
         + jnp.dot(xh, wl_ref[...], preferred_element_type=jnp.float32))
    yg = y[:, :D]
    yv = y[:, D:2 * D]
    ye = y[:, 2 * D:]
    sm = sm_ref[...]
    msg_ref[...] = jax.nn.silu(yg) * yv * sm
    enew_ref[...] = e + jax.nn.silu(ye)


def _combine(hs, hd, e, smooth, Wh, Wlo):
    return pl.pallas_call(
        _combine_body,
        grid=(E // B_EDGE,),
        in_specs=[
            pl.BlockSpec((B_EDGE, D), lambda i: (i, 0)),
            pl.BlockSpec((B_EDGE, D), lambda i: (i, 0)),
            pl.BlockSpec((B_EDGE, D), lambda i: (i, 0)),
            pl.BlockSpec((B_EDGE, 1), lambda i: (i, 0)),
            pl.BlockSpec((3 * D, 3 * D), lambda i: (0, 0)),
            pl.BlockSpec((3 * D, 3 * D), lambda i: (0, 0)),
        ],
        out_specs=[
            pl.BlockSpec((B_EDGE, D), lambda i: (i, 0)),
            pl.BlockSpec((B_EDGE, D), lambda i: (i, 0)),
        ],
        out_shape=[
            jax.ShapeDtypeStruct((E, D), jnp.float32),
            jax.ShapeDtypeStruct((E, D), jnp.float32),
        ],
    )(hs, hd, e, smooth, Wh, Wlo)


def _update_body(h_ref, p_ref, hn_ref):
    hn_ref[...] = h_ref[...] + p_ref[0] + p_ref[1]


def _update(h, parts):
    return pl.pallas_call(
        _update_body,
        grid=(N // B_NODE,),
        in_specs=[
            pl.BlockSpec((B_NODE, D), lambda i: (i, 0)),
            pl.BlockSpec((NC, B_NODE, D), lambda i: (0, i, 0)),
        ],
        out_specs=pl.BlockSpec((B_NODE, D), lambda i: (i, 0)),
        out_shape=jax.ShapeDtypeStruct((N, D), jnp.float32),
    )(h, parts)


def _head_body(h_ref, bid_ref, gamma_ref, w1_ref, w2_ref, out_ref, esum, csum):
    i = pl.program_id(0)

    @pl.when(i == 0)
    def _():
        esum[...] = jnp.zeros_like(esum)
        csum[...] = jnp.zeros_like(csum)

    h = h_ref[...]
    ms = jnp.mean(h * h, axis=1, keepdims=True)
    hn = h * lax.rsqrt(ms + 1e-6) * gamma_ref[0:1, :]
    t = jax.nn.silu(jnp.dot(hn, w1_ref[...], preferred_element_type=jnp.float32))
    ea = jnp.sum(t * w2_ref[0:1, :], axis=1, keepdims=True)
    mask = (bid_ref[...] == lax.broadcasted_iota(jnp.int32, (h.shape[0], G), 1))
    esum[...] += jnp.sum(jnp.where(mask, ea, 0.0), axis=0, keepdims=True)
    csum[...] += jnp.sum(mask.astype(jnp.float32), axis=0, keepdims=True)

    @pl.when(i == pl.num_programs(0) - 1)
    def _():
        out_ref[...] = esum[...] / jnp.maximum(csum[...], 1.0)


def _head(h, batch_ids2d, gamma2d, W1, W2row):
    return pl.pallas_call(
        _head_body,
        grid=(N // B_NODE,),
        in_specs=[
            pl.BlockSpec((B_NODE, D), lambda i: (i, 0)),
            pl.BlockSpec((B_NODE, 1), lambda i: (i, 0)),
            pl.BlockSpec((1, D), lambda i: (0, 0)),
            pl.BlockSpec((D, D), lambda i: (0, 0)),
            pl.BlockSpec((1, D), lambda i: (0, 0)),
        ],
        out_specs=pl.BlockSpec((1, G), lambda i: (0, 0)),
        out_shape=jax.ShapeDtypeStruct((1, G), jnp.float32),
        scratch_shapes=[
            pltpu.VMEM((1, G), jnp.float32),
            pltpu.VMEM((1, G), jnp.float32),
        ],
    )(h, batch_ids2d, gamma2d, W1, W2row)


def _force_body(e_ref, w_ref, f1_ref, f2_ref, fv_ref, fvn_ref):
    t = jax.nn.silu(jnp.dot(e_ref[...], f1_ref[...],
                            preferred_element_type=jnp.float32))
    fs = jnp.sum(t * f2_ref[0:1, :], axis=1, keepdims=True)
    fvec = fs * w_ref[...]
    z = jnp.zeros((fvec.shape[0], FPAD - 3), jnp.float32)
    fv = jnp.concatenate([fvec, z], axis=1)
    fv_ref[...] = fv
    fvn_ref[...] = -fv


def _force_head(e, w, F1, F2row):
    return pl.pallas_call(
        _force_body,
        grid=(E // B_EDGE,),
        in_specs=[
            pl.BlockSpec((B_EDGE, D), lambda i: (i, 0)),
            pl.BlockSpec((B_EDGE, 3), lambda i: (i, 0)),
            pl.BlockSpec((D, D), lambda i: (0, 0)),
            pl.BlockSpec((1, D), lambda i: (0, 0)),
        ],
        out_specs=[
            pl.BlockSpec((B_EDGE, FPAD), lambda i: (i, 0)),
            pl.BlockSpec((B_EDGE, FPAD), lambda i: (i, 0)),
        ],
        out_shape=[
            jax.ShapeDtypeStruct((E, FPAD), jnp.float32),
            jax.ShapeDtypeStruct((E, FPAD), jnp.float32),
        ],
    )(e, w, F1, F2row)


def _fassemble_body(p_ref, f_ref):
    f_ref[...] = (p_ref[0] + p_ref[1])[:, :3]


def _fassemble(parts):
    return pl.pallas_call(
        _fassemble_body,
        grid=(N // B_NODE,),
        in_specs=[pl.BlockSpec((NC, B_NODE, FPAD), lambda i: (0, i, 0))],
        out_specs=pl.BlockSpec((B_NODE, 3), lambda i: (i, 0)),
        out_shape=jax.ShapeDtypeStruct((N, 3), jnp.float32),
    )(parts)


def kernel(atomic_numbers, edge_index, edge_vec, batch_ids, atom_table,
           W_rbf, Wg, Wv, We, gamma, W1, W2, F1, F2):
    src = edge_index[0]
    dst = edge_index[1]

    e, smooth, w = _edge_init(edge_vec, W_rbf)
    h = _embed(atomic_numbers.astype(jnp.int32).reshape(N, 1), atom_table)

    Wcat = jnp.concatenate([Wg, Wv, We], axis=2)
    Wcat_h = Wcat.astype(jnp.bfloat16)
    Wcat_l = (Wcat - Wcat_h.astype(jnp.float32)).astype(jnp.bfloat16)

    zeros_nd = jnp.zeros((N, D), jnp.float32)
    for l in range(L):
        hs, hd = _sc_gather(h, src, dst)
        msg, e = _combine(hs, hd, e, smooth, Wcat_h[l], Wcat_l[l])
        parts = _sc_scatter(msg, dst, zeros_nd)
        h = _update(h, parts)

    energy = _head(h, batch_ids.astype(jnp.int32).reshape(N, 1),
                   gamma.reshape(1, D), W1, W2.reshape(1, D))[0]

    fv, fvn = _force_head(e, w, F1, F2.reshape(1, D))
    zeros_n8 = jnp.zeros((N, FPAD), jnp.float32)
    fparts = _sc_fscatter(fv, fvn, src, dst, zeros_n8)
    forces = _fassemble(fparts)

    return (energy, forces)


# speedup vs baseline: 2.3519x; 1.0324x over previous
import functools

import jax
import jax.numpy as jnp
from jax import lax
from jax.experimental import pallas as pl
from jax.experimental.pallas import tpu as pltpu
from jax.experimental.pallas import tpu_sc as plsc

N = 10000
E = 320000
G = 16
D = 128
R = 7
L = 6
CUT = 6.0

NC = 2
NS = 16
NW = NC * NS
CHG = 200
CH = 200
FPAD = 128


def _gather_body(table, src, dst, hs_out, hd_out,
                 idx0, idx1, rows0, rows1, g0, g1, w0, w1):
    wid = lax.axis_index("c") * NS + lax.axis_index("s")
    per_w = E // NW
    base = wid * per_w
    nch = per_w // CHG
    idxs = (idx0, idx1)
    rows = (rows0, rows1)
    gsem = (g0, g1)
    wsem = (w0, w1)

    def one(idx_hbm, out_hbm):
        for b in range(2):
            pltpu.sync_copy(idx_hbm.at[pl.ds(base + b * CHG, CHG)], idxs[b])
            pltpu.async_copy(table.at[idxs[b]], rows[b], gsem[b])

        def pair(jj, _):
            for b in range(2):
                i = jj * 2 + b
                off = base + i * CHG
                pltpu.make_async_copy(table.at[idxs[b]], rows[b], gsem[b]).wait()
                pltpu.async_copy(rows[b], out_hbm.at[pl.ds(off, CHG), :], wsem[b])

                @pl.when(i + 2 < nch)
                def _():
                    noff = base + (i + 2) * CHG
                    pltpu.sync_copy(idx_hbm.at[pl.ds(noff, CHG)], idxs[b])

                pltpu.make_async_copy(
                    rows[b], out_hbm.at[pl.ds(off, CHG), :], wsem[b]).wait()

                @pl.when(i + 2 < nch)
                def _():
                    pltpu.async_copy(table.at[idxs[b]], rows[b], gsem[b])
            return 0
        lax.fori_loop(0, nch // 2, pair, 0)

    one(src, hs_out)
    one(dst, hd_out)


@functools.lru_cache(maxsize=None)
def _sc_kernels():
    mesh = plsc.VectorSubcoreMesh(core_axis_name="c", subcore_axis_name="s",
                                  num_cores=NC, num_subcores=NS)
    gather = pl.kernel(
        _gather_body,
        out_type=(
            jax.ShapeDtypeStruct((E, D), jnp.float32),
            jax.ShapeDtypeStruct((E, D), jnp.float32),
        ),
        mesh=mesh,
        scratch_types=[
            pltpu.VMEM((CHG,), jnp.int32),
            pltpu.VMEM((CHG,), jnp.int32),
            pltpu.VMEM((CHG, D), jnp.float32),
            pltpu.VMEM((CHG, D), jnp.float32),
            pltpu.SemaphoreType.DMA,
            pltpu.SemaphoreType.DMA,
            pltpu.SemaphoreType.DMA,
            pltpu.SemaphoreType.DMA,
        ],
    )
    scatter = pl.kernel(
        _scatter_body,
        out_type=jax.ShapeDtypeStruct((NC, N, D), jnp.float32),
        mesh=mesh,
        scratch_types=[
            pltpu.VMEM_SHARED((N, D), jnp.float32),
            pltpu.VMEM((CH,), jnp.int32),
            pltpu.VMEM((CH, D), jnp.float32),
        ],
    )
    fscatter = pl.kernel(
        _fscatter_body,
        out_type=jax.ShapeDtypeStruct((NC, N, FPAD), jnp.float32),
        mesh=mesh,
        scratch_types=[
            pltpu.VMEM_SHARED((N, FPAD), jnp.float32),
            pltpu.VMEM((CH,), jnp.int32),
            pltpu.VMEM((CH, FPAD), jnp.float32),
        ],
    )
    return gather, scatter, fscatter


def _sc_gather(table, src, dst):
    return _sc_kernels()[0](table, src, dst)


def _scatter_body(msg, dst, zeros, out, acc, idx_v, rows_v):
    cid = lax.axis_index("c")
    sid = lax.axis_index("s")

    @pl.when(sid == 0)
    def _():
        pltpu.sync_copy(zeros, acc)

    plsc.subcore_barrier()

    per_w = (E // NC) // NS
    base = cid * (E // NC) + sid * per_w

    def step(i, _):
        off = base + i * CH
        pltpu.sync_copy(dst.at[pl.ds(off, CH)], idx_v)
        pltpu.sync_copy(msg.at[pl.ds(off, CH), :], rows_v)
        pltpu.sync_copy(rows_v, acc.at[idx_v], add=True)
        return 0
    lax.fori_loop(0, per_w // CH, step, 0)

    plsc.subcore_barrier()

    @pl.when(sid == 0)
    def _():
        pltpu.sync_copy(acc, out.at[cid])


def _sc_scatter(msg, dst, zeros):
    return _sc_kernels()[1](msg, dst, zeros)


def _fscatter_body(fv, fvn, src, dst, zeros, out, acc, idx_v, rows_v):
    cid = lax.axis_index("c")
    sid = lax.axis_index("s")

    @pl.when(sid == 0)
    def _():
        pltpu.sync_copy(zeros, acc)

    plsc.subcore_barrier()

    per_w = (E // NC) // NS
    base = cid * (E // NC) + sid * per_w

    def step(i, _):
        off = base + i * CH
        pltpu.sync_copy(dst.at[pl.ds(off, CH)], idx_v)
        pltpu.sync_copy(fv.at[pl.ds(off, CH), :], rows_v)
        pltpu.sync_copy(rows_v, acc.at[idx_v], add=True)
        pltpu.sync_copy(src.at[pl.ds(off, CH)], idx_v)
        pltpu.sync_copy(fvn.at[pl.ds(off, CH), :], rows_v)
        pltpu.sync_copy(rows_v, acc.at[idx_v], add=True)
        return 0
    lax.fori_loop(0, per_w // CH, step, 0)

    plsc.subcore_barrier()

    @pl.when(sid == 0)
    def _():
        pltpu.sync_copy(acc, out.at[cid])


def _sc_fscatter(fv, fvn, src, dst, zeros):
    return _sc_kernels()[2](fv, fvn, src, dst, zeros)


B_EDGE = 2000
B_NODE = 2000


def _edge_init_body(ev_ref, wr_ref, e0_ref, sm_ref, w_ref):
    ev = ev_ref[...]
    d2 = jnp.sum(ev * ev, axis=1, keepdims=True)
    d = jnp.sqrt(d2 + 1e-12)
    u = d / CUT
    u2 = u * u
    u4 = u2 * u2
    u8 = u4 * u4
    env = 1.0 + (-45.0) * u8 + 80.0 * u8 * u + (-36.0) * u8 * u2
    smooth = jnp.where(u < 1.0, env, 0.0)
    dinv = 1.0 / (d + 1e-8)
    k = lax.broadcasted_iota(jnp.int32, (ev.shape[0], R), 1).astype(jnp.float32) + 1.0
    s = jnp.sin(k * (jnp.pi * u))
    rbf = (jnp.sqrt(2.0 / CUT) * dinv * smooth) * s
    acc = jnp.zeros((ev.shape[0], D), jnp.float32)
    for r in range(R):
        acc = acc + rbf[:, r:r + 1] * wr_ref[r:r + 1, :]
    e0_ref[...] = acc
    sm_ref[...] = smooth
    w_ref[...] = ev * dinv * smooth


def _edge_init(edge_vec, W_rbf):
    return pl.pallas_call(
        _edge_init_body,
        grid=(E // B_EDGE,),
        in_specs=[
            pl.BlockSpec((B_EDGE, 3), lambda i: (i, 0)),
            pl.BlockSpec((R, D), lambda i: (0, 0)),
        ],
        out_specs=[
            pl.BlockSpec((B_EDGE, D), lambda i: (i, 0)),
            pl.BlockSpec((B_EDGE, 1), lambda i: (i, 0)),
            pl.BlockSpec((B_EDGE, 3), lambda i: (i, 0)),
        ],
        out_shape=[
            jax.ShapeDtypeStruct((E, D), jnp.float32),
            jax.ShapeDtypeStruct((E, 1), jnp.float32),
            jax.ShapeDtypeStruct((E, 3), jnp.float32),
        ],
    )(edge_vec, W_rbf)


def _embed_body(an_ref, tab_ref, h_ref):
    an = an_ref[...]
    oh = (an == lax.broadcasted_iota(jnp.int32, (an.shape[0], 95), 1))
    h_ref[...] = jnp.dot(oh.astype(jnp.float32), tab_ref[...],
                         preferred_element_type=jnp.float32)


def _embed(atomic_numbers2d, atom_table):
    return pl.pallas_call(
        _embed_body,
        grid=(N // B_NODE,),
        in_specs=[
            pl.BlockSpec((B_NODE, 1), lambda i: (i, 0)),
            pl.BlockSpec((95, D), lambda i: (0, 0)),
        ],
        out_specs=pl.BlockSpec((B_NODE, D), lambda i: (i, 0)),
        out_shape=jax.ShapeDtypeStruct((N, D), jnp.float32),
    )(atomic_numbers2d, atom_table)


def _combine_body(hs_ref, hd_ref, e_ref, sm_ref, wh_ref, wl_ref, msg_ref, enew_ref):
    e = e_ref[...]
    x = jnp.concatenate([hs_ref[...], hd_ref[...], e], axis=1)
    xh = x.astype(jnp.bfloat16)
    xl = (x - xh.astype(jnp.float32)).astype(jnp.bfloat16)
    wh = wh_ref[...]
    y = (jnp.dot(xh, wh, preferred_element_type=jnp.float32)
         + jnp.dot(xl, wh, preferred_element_type=jnp.float32)
         + jnp.dot(xh, wl_ref[...], preferred_element_type=jnp.float32))
    yg = y[:, :D]
    yv = y[:, D:2 * D]
    ye = y[:, 2 * D:]
    sm = sm_ref[...]
    msg_ref[...] = jax.nn.silu(yg) * yv * sm
    enew_ref[...] = e + jax.nn.silu(ye)


def _combine(hs, hd, e, smooth, Wh, Wlo):
    return pl.pallas_call(
        _combine_body,
        grid=(E // B_EDGE,),
        in_specs=[
            pl.BlockSpec((B_EDGE, D), lambda i: (i, 0)),
            pl.BlockSpec((B_EDGE, D), lambda i: (i, 0)),
            pl.BlockSpec((B_EDGE, D), lambda i: (i, 0)),
            pl.BlockSpec((B_EDGE, 1), lambda i: (i, 0)),
            pl.BlockSpec((3 * D, 3 * D), lambda i: (0, 0)),
            pl.BlockSpec((3 * D, 3 * D), lambda i: (0, 0)),
        ],
        out_specs=[
            pl.BlockSpec((B_EDGE, D), lambda i: (i, 0)),
            pl.BlockSpec((B_EDGE, D), lambda i: (i, 0)),
        ],
        out_shape=[
            jax.ShapeDtypeStruct((E, D), jnp.float32),
            jax.ShapeDtypeStruct((E, D), jnp.float32),
        ],
    )(hs, hd, e, smooth, Wh, Wlo)


def _update_body(h_ref, p_ref, hn_ref):
    hn_ref[...] = h_ref[...] + p_ref[0] + p_ref[1]


def _update(h, parts):
    return pl.pallas_call(
        _update_body,
        grid=(N // B_NODE,),
        in_specs=[
            pl.BlockSpec((B_NODE, D), lambda i: (i, 0)),
            pl.BlockSpec((NC, B_NODE, D), lambda i: (0, i, 0)),
        ],
        out_specs=pl.BlockSpec((B_NODE, D), lambda i: (i, 0)),
        out_shape=jax.ShapeDtypeStruct((N, D), jnp.float32),
    )(h, parts)


def _head_body(h_ref, bid_ref, gamma_ref, w1_ref, w2_ref, out_ref, esum, csum):
    i = pl.program_id(0)

    @pl.when(i == 0)
    def _():
        esum[...] = jnp.zeros_like(esum)
        csum[...] = jnp.zeros_like(csum)

    h = h_ref[...]
    ms = jnp.mean(h * h, axis=1, keepdims=True)
    hn = h * lax.rsqrt(ms + 1e-6) * gamma_ref[0:1, :]
    t = jax.nn.silu(jnp.dot(hn, w1_ref[...], preferred_element_type=jnp.float32))
    ea = jnp.sum(t * w2_ref[0:1, :], axis=1, keepdims=True)
    mask = (bid_ref[...] == lax.broadcasted_iota(jnp.int32, (h.shape[0], G), 1))
    esum[...] += jnp.sum(jnp.where(mask, ea, 0.0), axis=0, keepdims=True)
    csum[...] += jnp.sum(mask.astype(jnp.float32), axis=0, keepdims=True)

    @pl.when(i == pl.num_programs(0) - 1)
    def _():
        out_ref[...] = esum[...] / jnp.maximum(csum[...], 1.0)


def _head(h, batch_ids2d, gamma2d, W1, W2row):
    return pl.pallas_call(
        _head_body,
        grid=(N // B_NODE,),
        in_specs=[
            pl.BlockSpec((B_NODE, D), lambda i: (i, 0)),
            pl.BlockSpec((B_NODE, 1), lambda i: (i, 0)),
            pl.BlockSpec((1, D), lambda i: (0, 0)),
            pl.BlockSpec((D, D), lambda i: (0, 0)),
            pl.BlockSpec((1, D), lambda i: (0, 0)),
        ],
        out_specs=pl.BlockSpec((1, G), lambda i: (0, 0)),
        out_shape=jax.ShapeDtypeStruct((1, G), jnp.float32),
        scratch_shapes=[
            pltpu.VMEM((1, G), jnp.float32),
            pltpu.VMEM((1, G), jnp.float32),
        ],
    )(h, batch_ids2d, gamma2d, W1, W2row)


def _force_body(e_ref, w_ref, f1_ref, f2_ref, fv_ref, fvn_ref):
    t = jax.nn.silu(jnp.dot(e_ref[...], f1_ref[...],
                            preferred_element_type=jnp.float32))
    fs = jnp.sum(t * f2_ref[0:1, :], axis=1, keepdims=True)
    fvec = fs * w_ref[...]
    z = jnp.zeros((fvec.shape[0], FPAD - 3), jnp.float32)
    fv = jnp.concatenate([fvec, z], axis=1)
    fv_ref[...] = fv
    fvn_ref[...] = -fv


def _force_head(e, w, F1, F2row):
    return pl.pallas_call(
        _force_body,
        grid=(E // B_EDGE,),
        in_specs=[
            pl.BlockSpec((B_EDGE, D), lambda i: (i, 0)),
            pl.BlockSpec((B_EDGE, 3), lambda i: (i, 0)),
            pl.BlockSpec((D, D), lambda i: (0, 0)),
            pl.BlockSpec((1, D), lambda i: (0, 0)),
        ],
        out_specs=[
            pl.BlockSpec((B_EDGE, FPAD), lambda i: (i, 0)),
            pl.BlockSpec((B_EDGE, FPAD), lambda i: (i, 0)),
        ],
        out_shape=[
            jax.ShapeDtypeStruct((E, FPAD), jnp.float32),
            jax.ShapeDtypeStruct((E, FPAD), jnp.float32),
        ],
    )(e, w, F1, F2row)


def _fassemble_body(p_ref, f_ref):
    f_ref[...] = (p_ref[0] + p_ref[1])[:, :3]


def _fassemble(parts):
    return pl.pallas_call(
        _fassemble_body,
        grid=(N // B_NODE,),
        in_specs=[pl.BlockSpec((NC, B_NODE, FPAD), lambda i: (0, i, 0))],
        out_specs=pl.BlockSpec((B_NODE, 3), lambda i: (i, 0)),
        out_shape=jax.ShapeDtypeStruct((N, 3), jnp.float32),
    )(parts)


def kernel(atomic_numbers, edge_index, edge_vec, batch_ids, atom_table,
           W_rbf, Wg, Wv, We, gamma, W1, W2, F1, F2):
    src = edge_index[0]
    dst = edge_index[1]

    e, smooth, w = _edge_init(edge_vec, W_rbf)
    h = _embed(atomic_numbers.astype(jnp.int32).reshape(N, 1), atom_table)

    Wcat = jnp.concatenate([Wg, Wv, We], axis=2)
    Wcat_h = Wcat.astype(jnp.bfloat16)
    Wcat_l = (Wcat - Wcat_h.astype(jnp.float32)).astype(jnp.bfloat16)

    zeros_nd = jnp.zeros((N, D), jnp.float32)
    for l in range(L):
        hs, hd = _sc_gather(h, src, dst)
        msg, e = _combine(hs, hd, e, smooth, Wcat_h[l], Wcat_l[l])
        parts = _sc_scatter(msg, dst, zeros_nd)
        h = _update(h, parts)

    energy = _head(h, batch_ids.astype(jnp.int32).reshape(N, 1),
                   gamma.reshape(1, D), W1, W2.reshape(1, D))[0]

    fv, fvn = _force_head(e, w, F1, F2.reshape(1, D))
    zeros_n8 = jnp.zeros((N, FPAD), jnp.float32)
    fparts = _sc_fscatter(fv, fvn, src, dst, zeros_n8)
    forces = _fassemble(fparts)

    return (energy, forces)
